# Initial kernel scaffold; baseline (speedup 1.0000x reference)
import jax
import jax.numpy as jnp
from jax.experimental import pallas as pl


def kernel(x):
    raise NotImplementedError("write your pallas kernel here")


# speedup vs baseline: 4.8874x; 4.8874x over previous
import jax
import jax.numpy as jnp
from jax.experimental import pallas as pl
from jax.experimental.pallas import tpu as pltpu

K = 64
ROWS = 128
COLS = 32768
BLOCK_ROWS = 8


def _topk_mask_body(x_ref, o_ref):
    INT_MIN = jnp.int32(-(2 ** 31))
    x = x_ref[...]
    b = jax.lax.bitcast_convert_type(x, jnp.int32)
    m = jnp.where(b >= 0, b, jnp.bitwise_xor(jnp.bitwise_not(b), INT_MIN))

    t = jnp.zeros((x.shape[0], 1), jnp.int32)
    for bit in range(31, -1, -1):
        cand = t | (INT_MIN if bit == 31 else jnp.int32(1 << bit))
        csign = cand ^ INT_MIN
        cnt = jnp.sum((m >= csign).astype(jnp.int32), axis=1, keepdims=True)
        t = jnp.where(cnt >= K, cand, t)
    m_t = t ^ INT_MIN

    keep = (m >= m_t) & (x > 0.0)
    o_ref[...] = jnp.where(keep, x, 0.0)


def kernel(x):
    grid = (ROWS // BLOCK_ROWS,)
    return pl.pallas_call(
        _topk_mask_body,
        grid=grid,
        in_specs=[pl.BlockSpec((BLOCK_ROWS, COLS), lambda i: (i, 0))],
        out_specs=pl.BlockSpec((BLOCK_ROWS, COLS), lambda i: (i, 0)),
        out_shape=jax.ShapeDtypeStruct((ROWS, COLS), jnp.float32),
        compiler_params=pltpu.CompilerParams(
            dimension_semantics=("arbitrary",),
        ),
    )(x)
